# baseline (speedup 1.0000x reference)
import jax
import jax.numpy as jnp
from jax import lax
from jax.experimental import pallas as pl
from jax.experimental.pallas import tpu as pltpu
from jax.experimental.pallas import tpu_sc as plsc

N = 10000
E = 320000
D = 128
G = 128
NUM_LAYER = 5
NUM_CLASS = 6
BN_EPS = 1e-5

NB = 400
NGRID = N // NB

NC = 2
NS = 16
LANES = 16
NW = NC * NS
CH = 128
CPT = 80
BLK = 8
NBLK = CPT // BLK
E_PAD = NW * CPT * CH
N_PAD = 10240
RPT = N_PAD // NS


def _edge_body(hcur_hbm, eidx_hbm, etab_hbm, out_hbm,
               acc, islab, buf0, buf1, etab_v, gsem0, gsem1, isem,
               ssem0, ssem1):
    c = lax.axis_index("c")
    s = lax.axis_index("s")
    wid = c * NS + s

    pltpu.sync_copy(etab_hbm, etab_v)

    zeros16 = jnp.zeros((LANES,), jnp.float32)

    def zrow(r, carry):
        for jj in range(D // LANES):
            buf0[r, pl.ds(jj * LANES, LANES)] = zeros16
        return carry

    lax.fori_loop(0, CH, zrow, 0)
    base = s * RPT
    for j in range(RPT // CH):
        pltpu.sync_copy(buf0, acc.at[pl.ds(base + j * CH, CH)])
    plsc.subcore_barrier()

    pltpu.sync_copy(eidx_hbm.at[wid, 0], islab.at[0])
    pltpu.async_copy(hcur_hbm.at[islab.at[0, 0, 0]], buf0, gsem0)

    def b_body(b, carry):
        p = b % 2
        q = 1 - p

        def jb2_body(jb2, carry2):
            for k in range(2):
                jb = jb2 * 2 + k
                buf = buf0 if k == 0 else buf1
                obuf = buf1 if k == 0 else buf0
                gsem = gsem0 if k == 0 else gsem1
                ogsem = gsem1 if k == 0 else gsem0
                pltpu.make_async_copy(
                    hcur_hbm.at[islab.at[p, 0, jb]], buf, gsem).wait()

                @pl.when(jb < BLK - 1)
                def _():
                    pltpu.async_copy(
                        hcur_hbm.at[islab.at[p, 0, jb + 1]], obuf, ogsem)

                @pl.when((jb == BLK - 1) & (b + 1 < NBLK))
                def _():
                    pltpu.make_async_copy(
                        eidx_hbm.at[wid, b + 1], islab.at[q], isem).wait()
                    pltpu.async_copy(
                        hcur_hbm.at[islab.at[q, 0, 0]], obuf, ogsem)

                @pl.when((jb == 1) & (b + 1 < NBLK))
                def _():
                    pltpu.async_copy(eidx_hbm.at[wid, b + 1], islab.at[q], isem)

                @plsc.parallel_loop(0, CH // LANES, 1, unroll=2)
                def cgrp(g):
                    civ = islab[p, 2, jb, pl.ds(g * LANES, LANES)]
                    for e in range(LANES):
                        ci = civ[e]
                        r = g * LANES + e
                        for jj in range(D // LANES):
                            sl = pl.ds(jj * LANES, LANES)
                            buf[r, sl] = jnp.maximum(
                                buf[r, sl] + etab_v[ci, sl], 0.0)

                pltpu.sync_copy(buf, acc.at[islab.at[p, 1, jb]], add=True)
            return carry2

        lax.fori_loop(0, BLK // 2, jb2_body, 0)
        return carry

    lax.fori_loop(0, NBLK, b_body, 0)
    plsc.subcore_barrier()
    for j in range(RPT // CH):
        sl = pl.ds(base + j * CH, CH)
        pltpu.sync_copy(acc.at[sl], out_hbm.at[c, sl])


_EDGE_CALL_CACHE = []


def _make_edge_call():
    if _EDGE_CALL_CACHE:
        return _EDGE_CALL_CACHE[0]
    call = pl.kernel(
        _edge_body,
        out_type=jax.ShapeDtypeStruct((NC, N_PAD, D), jnp.float32),
        mesh=plsc.VectorSubcoreMesh(core_axis_name="c", subcore_axis_name="s",
                                    num_cores=NC, num_subcores=NS),
        scratch_types=[
            pltpu.VMEM_SHARED((N_PAD, D), jnp.float32),
            pltpu.VMEM((2, 3, BLK, CH), jnp.int32),
            pltpu.VMEM((CH, D), jnp.float32),
            pltpu.VMEM((CH, D), jnp.float32),
            pltpu.VMEM((64, D), jnp.float32),
            pltpu.SemaphoreType.DMA,
            pltpu.SemaphoreType.DMA,
            pltpu.SemaphoreType.DMA,
            pltpu.SemaphoreType.DMA,
            pltpu.SemaphoreType.DMA,
        ],
    )
    _EDGE_CALL_CACHE.append(call)
    return call


def _edge_agg(hcur, eidx, etab_l):
    return _make_edge_call()(hcur, eidx, etab_l)


def _enc_body(x_ref, b_ref, aemb_ref, bflat_ref, vne_ref,
              h0_ref, hcur0_ref, bmat_ref, etab_ref):
    i = pl.program_id(0)
    xb = x_ref[...]
    iota64 = lax.broadcasted_iota(jnp.int32, (NB, 64), 1)
    h = jnp.zeros((NB, D), jnp.float32)
    for f in range(9):
        oh = (xb[:, f][:, None] == iota64).astype(jnp.float32)
        h = h + jnp.dot(oh, aemb_ref[f], preferred_element_type=jnp.float32)
    h0_ref[...] = h
    hcur0_ref[...] = h + vne_ref[...]
    bb = b_ref[...]
    iotaG = lax.broadcasted_iota(jnp.int32, (NB, G), 1)
    bmat_ref[...] = (bb == iotaG).astype(jnp.float32)

    @pl.when(i == 0)
    def _():
        r = lax.broadcasted_iota(jnp.int32, (NUM_LAYER * 64, 120), 0)
        col = lax.broadcasted_iota(jnp.int32, (NUM_LAYER * 64, 120), 1)
        lr, cc = r // 64, r % 64
        lc, f, dd = col // 24, (col % 24) // 8, col % 8
        sel = ((lr == lc) & (((cc >> (2 * f)) & 3) == dd)).astype(jnp.float32)
        etab_ref[...] = jnp.dot(sel, bflat_ref[...],
                                preferred_element_type=jnp.float32)


def _enc_call(x, batch2d, aemb, bflat, vne):
    return pl.pallas_call(
        _enc_body,
        grid=(NGRID,),
        in_specs=[
            pl.BlockSpec((NB, 9), lambda i: (i, 0)),
            pl.BlockSpec((NB, 1), lambda i: (i, 0)),
            pl.BlockSpec((9, 64, D), lambda i: (0, 0, 0)),
            pl.BlockSpec((120, D), lambda i: (0, 0)),
            pl.BlockSpec((1, D), lambda i: (0, 0)),
        ],
        out_specs=[
            pl.BlockSpec((NB, D), lambda i: (i, 0)),
            pl.BlockSpec((NB, D), lambda i: (i, 0)),
            pl.BlockSpec((NB, G), lambda i: (i, 0)),
            pl.BlockSpec((NUM_LAYER * 64, D), lambda i: (0, 0)),
        ],
        out_shape=[
            jax.ShapeDtypeStruct((N, D), jnp.float32),
            jax.ShapeDtypeStruct((N, D), jnp.float32),
            jax.ShapeDtypeStruct((N, G), jnp.float32),
            jax.ShapeDtypeStruct((NUM_LAYER * 64, D), jnp.float32),
        ],
    )(x, batch2d, aemb, bflat, vne)


def _vn_body(h_ref, bm_ref, vn_ref, w1_ref, b1_ref, g1_ref, t1_ref,
             w2_ref, b2_ref, g2_ref, t2_ref, out_ref, acc_ref):
    i = pl.program_id(0)

    @pl.when(i == 0)
    def _():
        acc_ref[...] = jnp.zeros_like(acc_ref)

    acc_ref[...] += lax.dot_general(bm_ref[...], h_ref[...],
                                    (((0,), (0,)), ((), ())),
                                    preferred_element_type=jnp.float32)

    @pl.when(i == NGRID - 1)
    def _():
        vt = acc_ref[...] + vn_ref[...]
        vt = jnp.dot(vt, w1_ref[...], preferred_element_type=jnp.float32) + b1_ref[...]
        vt = jnp.maximum(vt * g1_ref[...] + t1_ref[...], 0.0)
        vt = jnp.dot(vt, w2_ref[...], preferred_element_type=jnp.float32) + b2_ref[...]
        vt = jnp.maximum(vt * g2_ref[...] + t2_ref[...], 0.0)
        out_ref[...] = vt


def _vn_call(h, bmat, vn, w1, b1, g1, t1, w2, b2, g2, t2):
    return pl.pallas_call(
        _vn_body,
        grid=(NGRID,),
        in_specs=[
            pl.BlockSpec((NB, D), lambda i: (i, 0)),
            pl.BlockSpec((NB, G), lambda i: (i, 0)),
            pl.BlockSpec((G, D), lambda i: (0, 0)),
            pl.BlockSpec((D, 2 * D), lambda i: (0, 0)),
            pl.BlockSpec((1, 2 * D), lambda i: (0, 0)),
            pl.BlockSpec((1, 2 * D), lambda i: (0, 0)),
            pl.BlockSpec((1, 2 * D), lambda i: (0, 0)),
            pl.BlockSpec((2 * D, D), lambda i: (0, 0)),
            pl.BlockSpec((1, D), lambda i: (0, 0)),
            pl.BlockSpec((1, D), lambda i: (0, 0)),
            pl.BlockSpec((1, D), lambda i: (0, 0)),
        ],
        out_specs=pl.BlockSpec((G, D), lambda i: (0, 0)),
        out_shape=jax.ShapeDtypeStruct((G, D), jnp.float32),
        scratch_shapes=[pltpu.VMEM((G, D), jnp.float32)],
    )(h, bmat, vn, w1, b1, g1, t1, w2, b2, g2, t2)


def _node_body(do_relu, hcur_ref, a0_ref, a1_ref, vnn_ref, bm_ref, eps_ref,
               w1_ref, b1_ref, g1_ref, t1_ref, w2_ref, b2_ref, g2_ref, t2_ref,
               hn_ref, hcn_ref):
    z = eps_ref[0, 0] * hcur_ref[...] + a0_ref[...] + a1_ref[...]
    z = jnp.dot(z, w1_ref[...], preferred_element_type=jnp.float32) + b1_ref[...]
    z = jnp.maximum(z * g1_ref[...] + t1_ref[...], 0.0)
    z = jnp.dot(z, w2_ref[...], preferred_element_type=jnp.float32) + b2_ref[...]
    z = z * g2_ref[...] + t2_ref[...]
    if do_relu:
        z = jnp.maximum(z, 0.0)
    hn_ref[...] = z
    hcn_ref[...] = z + jnp.dot(bm_ref[...], vnn_ref[...],
                               preferred_element_type=jnp.float32)


def _node_call(do_relu, hcur, a0, a1, vnn, bmat, epsp,
               w1, b1, g1, t1, w2, b2, g2, t2):
    import functools
    return pl.pallas_call(
        functools.partial(_node_body, do_relu),
        grid=(NGRID,),
        in_specs=[
            pl.BlockSpec((NB, D), lambda i: (i, 0)),
            pl.BlockSpec((NB, D), lambda i: (i, 0)),
            pl.BlockSpec((NB, D), lambda i: (i, 0)),
            pl.BlockSpec((G, D), lambda i: (0, 0)),
            pl.BlockSpec((NB, G), lambda i: (i, 0)),
            pl.BlockSpec((1, 1), lambda i: (0, 0)),
            pl.BlockSpec((D, 2 * D), lambda i: (0, 0)),
            pl.BlockSpec((1, 2 * D), lambda i: (0, 0)),
            pl.BlockSpec((1, 2 * D), lambda i: (0, 0)),
            pl.BlockSpec((1, 2 * D), lambda i: (0, 0)),
            pl.BlockSpec((2 * D, D), lambda i: (0, 0)),
            pl.BlockSpec((1, D), lambda i: (0, 0)),
            pl.BlockSpec((1, D), lambda i: (0, 0)),
            pl.BlockSpec((1, D), lambda i: (0, 0)),
        ],
        out_specs=[
            pl.BlockSpec((NB, D), lambda i: (i, 0)),
            pl.BlockSpec((NB, D), lambda i: (i, 0)),
        ],
        out_shape=[
            jax.ShapeDtypeStruct((N, D), jnp.float32),
            jax.ShapeDtypeStruct((N, D), jnp.float32),
        ],
    )(hcur, a0, a1, vnn, bmat, epsp, w1, b1, g1, t1, w2, b2, g2, t2)


def _final_body(h_ref, bm_ref, pw_ref, pb_ref, out_ref, accp_ref, accc_ref):
    i = pl.program_id(0)

    @pl.when(i == 0)
    def _():
        accp_ref[...] = jnp.zeros_like(accp_ref)
        accc_ref[...] = jnp.zeros_like(accc_ref)

    bm = bm_ref[...]
    accp_ref[...] += lax.dot_general(bm, h_ref[...], (((0,), (0,)), ((), ())),
                                     preferred_element_type=jnp.float32)
    accc_ref[...] += lax.dot_general(bm, jnp.ones((NB, 8), jnp.float32),
                                     (((0,), (0,)), ((), ())),
                                     preferred_element_type=jnp.float32)

    @pl.when(i == NGRID - 1)
    def _():
        cnt = jnp.maximum(accc_ref[...][:, :1], 1.0)
        hg = accp_ref[...] / cnt
        out_ref[...] = jnp.dot(hg, pw_ref[...],
                               preferred_element_type=jnp.float32) + pb_ref[...]


def _final_call(h, bmat, pw, pb):
    return pl.pallas_call(
        _final_body,
        grid=(NGRID,),
        in_specs=[
            pl.BlockSpec((NB, D), lambda i: (i, 0)),
            pl.BlockSpec((NB, G), lambda i: (i, 0)),
            pl.BlockSpec((D, 128), lambda i: (0, 0)),
            pl.BlockSpec((1, 128), lambda i: (0, 0)),
        ],
        out_specs=pl.BlockSpec((G, 128), lambda i: (0, 0)),
        out_shape=jax.ShapeDtypeStruct((G, 128), jnp.float32),
        scratch_shapes=[pltpu.VMEM((G, D), jnp.float32),
                        pltpu.VMEM((G, 8), jnp.float32)],
    )(h, bmat, pw, pb)


def kernel(x, edge_index, edge_attr, batch, params):
    p = params
    inv = (1.0 + BN_EPS) ** -0.5

    src = edge_index[0]
    dst = edge_index[1]
    ecls = edge_attr[:, 0] + 4 * edge_attr[:, 1] + 16 * edge_attr[:, 2]
    pad = E_PAD - E
    src_p = jnp.concatenate([src, jnp.zeros((pad,), jnp.int32)]).reshape(
        NW, NBLK, BLK, CH)
    dst_p = jnp.concatenate(
        [dst, N + (jnp.arange(pad, dtype=jnp.int32) % 128)]).reshape(
        NW, NBLK, BLK, CH)
    ecls_p = jnp.concatenate([ecls, jnp.zeros((pad,), jnp.int32)]).reshape(
        NW, NBLK, BLK, CH)
    eidx = jnp.stack([src_p, dst_p, ecls_p], axis=2)

    bflat = p['bond_emb'].reshape(NUM_LAYER * 3 * 8, D)
    vne = p['vn_emb'].reshape(1, D)
    batch2d = batch.reshape(N, 1)

    h, hcur, bmat, etab = _enc_call(x, batch2d, p['atom_emb'], bflat, vne)

    vn = jnp.broadcast_to(p['vn_emb'], (G, D))
    out = None
    for l in range(NUM_LAYER):
        aggp = _edge_agg(hcur, eidx, etab[64 * l:64 * (l + 1)])
        a0 = aggp[0, :N]
        a1 = aggp[1, :N]
        if l < NUM_LAYER - 1:
            vn = _vn_call(
                h, bmat, vn,
                p['vn_W1'][l], p['vn_b1'][l].reshape(1, -1),
                (p['vn_bn1_g'][l] * inv).reshape(1, -1),
                p['vn_bn1_b'][l].reshape(1, -1),
                p['vn_W2'][l], p['vn_b2'][l].reshape(1, -1),
                (p['vn_bn2_g'][l] * inv).reshape(1, -1),
                p['vn_bn2_b'][l].reshape(1, -1),
            )
        else:
            vn = jnp.zeros((G, D), jnp.float32)
        epsp = (1.0 + p['eps'][l]).reshape(1, 1)
        h, hcur = _node_call(
            l < NUM_LAYER - 1, hcur, a0, a1, vn, bmat, epsp,
            p['mlp_W1'][l], p['mlp_b1'][l].reshape(1, -1),
            (p['mlp_bn_g'][l] * inv).reshape(1, -1),
            p['mlp_bn_b'][l].reshape(1, -1),
            p['mlp_W2'][l], p['mlp_b2'][l].reshape(1, -1),
            (p['bn_g'][l] * inv).reshape(1, -1),
            p['bn_b'][l].reshape(1, -1),
        )

    pw = jnp.pad(p['pred_W'], ((0, 0), (0, 128 - NUM_CLASS)))
    pb = jnp.pad(p['pred_b'], (0, 128 - NUM_CLASS)).reshape(1, 128)
    out = _final_call(h, bmat, pw, pb)
    return out[:, :NUM_CLASS]


# speedup vs baseline: 1.4927x; 1.4927x over previous
import jax
import jax.numpy as jnp
from jax import lax
from jax.experimental import pallas as pl
from jax.experimental.pallas import tpu as pltpu
from jax.experimental.pallas import tpu_sc as plsc

N = 10000
E = 320000
D = 128
G = 128
NUM_LAYER = 5
NUM_CLASS = 6
BN_EPS = 1e-5

NB = 400
NGRID = N // NB

NC = 2
NS = 16
LANES = 16
NW = NC * NS
CH = 128
CPT = 80
BLK = 8
NBLK = CPT // BLK
E_PAD = NW * CPT * CH
N_PAD = 10240
RPT = N_PAD // NS


def _edge_body(hcur_hbm, eidx_hbm, etab_hbm, out_hbm,
               acc, islab, buf0, buf1, etab_v, gsem0, gsem1, isem,
               ssem0, ssem1):
    c = lax.axis_index("c")
    s = lax.axis_index("s")
    wid = c * NS + s

    pltpu.sync_copy(etab_hbm, etab_v)

    zeros16 = jnp.zeros((LANES,), jnp.float32)

    def zrow(r, carry):
        for jj in range(D // LANES):
            buf0[r, pl.ds(jj * LANES, LANES)] = zeros16
        return carry

    lax.fori_loop(0, CH, zrow, 0)
    base = s * RPT
    for j in range(RPT // CH):
        pltpu.sync_copy(buf0, acc.at[pl.ds(base + j * CH, CH)])
    plsc.subcore_barrier()

    pltpu.sync_copy(eidx_hbm.at[wid, 0], islab.at[0])
    pltpu.async_copy(hcur_hbm.at[islab.at[0, 0, 0]], buf0, gsem0)

    def b_body(b, carry):
        p = b % 2
        q = 1 - p

        def jb2_body(jb2, carry2):
            for k in range(2):
                jb = jb2 * 2 + k
                buf = buf0 if k == 0 else buf1
                obuf = buf1 if k == 0 else buf0
                gsem = gsem0 if k == 0 else gsem1
                ogsem = gsem1 if k == 0 else gsem0
                pltpu.make_async_copy(
                    hcur_hbm.at[islab.at[p, 0, jb]], buf, gsem).wait()

                @pl.when(jb < BLK - 1)
                def _():
                    pltpu.async_copy(
                        hcur_hbm.at[islab.at[p, 0, jb + 1]], obuf, ogsem)

                @pl.when((jb == BLK - 1) & (b + 1 < NBLK))
                def _():
                    pltpu.make_async_copy(
                        eidx_hbm.at[wid, b + 1], islab.at[q], isem).wait()
                    pltpu.async_copy(
                        hcur_hbm.at[islab.at[q, 0, 0]], obuf, ogsem)

                @pl.when((jb == 1) & (b + 1 < NBLK))
                def _():
                    pltpu.async_copy(eidx_hbm.at[wid, b + 1], islab.at[q], isem)

                @plsc.parallel_loop(0, CH // LANES, 1)
                def cgrp(g):
                    civ = islab[p, 2, jb, pl.ds(g * LANES, LANES)]
                    for e in range(LANES):
                        ci = civ[e]
                        r = g * LANES + e
                        for jj in range(D // LANES):
                            sl = pl.ds(jj * LANES, LANES)
                            buf[r, sl] = jnp.maximum(
                                buf[r, sl] + etab_v[ci, sl], 0.0)

                pltpu.sync_copy(buf, acc.at[islab.at[p, 1, jb]], add=True)
            return carry2

        lax.fori_loop(0, BLK // 2, jb2_body, 0)
        return carry

    lax.fori_loop(0, NBLK, b_body, 0)
    plsc.subcore_barrier()
    for j in range(RPT // CH):
        sl = pl.ds(base + j * CH, CH)
        pltpu.sync_copy(acc.at[sl], out_hbm.at[c, sl])


_EDGE_CALL_CACHE = []


def _make_edge_call():
    if _EDGE_CALL_CACHE:
        return _EDGE_CALL_CACHE[0]
    call = pl.kernel(
        _edge_body,
        out_type=jax.ShapeDtypeStruct((NC, N_PAD, D), jnp.float32),
        mesh=plsc.VectorSubcoreMesh(core_axis_name="c", subcore_axis_name="s",
                                    num_cores=NC, num_subcores=NS),
        scratch_types=[
            pltpu.VMEM_SHARED((N_PAD, D), jnp.float32),
            pltpu.VMEM((2, 3, BLK, CH), jnp.int32),
            pltpu.VMEM((CH, D), jnp.float32),
            pltpu.VMEM((CH, D), jnp.float32),
            pltpu.VMEM((64, D), jnp.float32),
            pltpu.SemaphoreType.DMA,
            pltpu.SemaphoreType.DMA,
            pltpu.SemaphoreType.DMA,
            pltpu.SemaphoreType.DMA,
            pltpu.SemaphoreType.DMA,
        ],
    )
    _EDGE_CALL_CACHE.append(call)
    return call


def _edge_agg(hcur, eidx, etab_l):
    return _make_edge_call()(hcur, eidx, etab_l)


def _enc_body(x_ref, b_ref, aemb_ref, bflat_ref, vne_ref,
              h0_ref, hcur0_ref, bmat_ref, etab_ref):
    i = pl.program_id(0)
    xb = x_ref[...]
    iota64 = lax.broadcasted_iota(jnp.int32, (NB, 64), 1)
    h = jnp.zeros((NB, D), jnp.float32)
    for f in range(9):
        oh = (xb[:, f][:, None] == iota64).astype(jnp.float32)
        h = h + jnp.dot(oh, aemb_ref[f], preferred_element_type=jnp.float32)
    h0_ref[...] = h
    hcur0_ref[...] = h + vne_ref[...]
    bb = b_ref[...]
    iotaG = lax.broadcasted_iota(jnp.int32, (NB, G), 1)
    bmat_ref[...] = (bb == iotaG).astype(jnp.float32)

    @pl.when(i == 0)
    def _():
        r = lax.broadcasted_iota(jnp.int32, (NUM_LAYER * 64, 120), 0)
        col = lax.broadcasted_iota(jnp.int32, (NUM_LAYER * 64, 120), 1)
        lr, cc = r // 64, r % 64
        lc, f, dd = col // 24, (col % 24) // 8, col % 8
        sel = ((lr == lc) & (((cc >> (2 * f)) & 3) == dd)).astype(jnp.float32)
        etab_ref[...] = jnp.dot(sel, bflat_ref[...],
                                preferred_element_type=jnp.float32)


def _enc_call(x, batch2d, aemb, bflat, vne):
    return pl.pallas_call(
        _enc_body,
        grid=(NGRID,),
        in_specs=[
            pl.BlockSpec((NB, 9), lambda i: (i, 0)),
            pl.BlockSpec((NB, 1), lambda i: (i, 0)),
            pl.BlockSpec((9, 64, D), lambda i: (0, 0, 0)),
            pl.BlockSpec((120, D), lambda i: (0, 0)),
            pl.BlockSpec((1, D), lambda i: (0, 0)),
        ],
        out_specs=[
            pl.BlockSpec((NB, D), lambda i: (i, 0)),
            pl.BlockSpec((NB, D), lambda i: (i, 0)),
            pl.BlockSpec((NB, G), lambda i: (i, 0)),
            pl.BlockSpec((NUM_LAYER * 64, D), lambda i: (0, 0)),
        ],
        out_shape=[
            jax.ShapeDtypeStruct((N, D), jnp.float32),
            jax.ShapeDtypeStruct((N, D), jnp.float32),
            jax.ShapeDtypeStruct((N, G), jnp.float32),
            jax.ShapeDtypeStruct((NUM_LAYER * 64, D), jnp.float32),
        ],
    )(x, batch2d, aemb, bflat, vne)


def _vn_body(h_ref, bm_ref, vn_ref, w1_ref, b1_ref, g1_ref, t1_ref,
             w2_ref, b2_ref, g2_ref, t2_ref, out_ref, acc_ref):
    i = pl.program_id(0)

    @pl.when(i == 0)
    def _():
        acc_ref[...] = jnp.zeros_like(acc_ref)

    acc_ref[...] += lax.dot_general(bm_ref[...], h_ref[...],
                                    (((0,), (0,)), ((), ())),
                                    preferred_element_type=jnp.float32)

    @pl.when(i == NGRID - 1)
    def _():
        vt = acc_ref[...] + vn_ref[...]
        vt = jnp.dot(vt, w1_ref[...], preferred_element_type=jnp.float32) + b1_ref[...]
        vt = jnp.maximum(vt * g1_ref[...] + t1_ref[...], 0.0)
        vt = jnp.dot(vt, w2_ref[...], preferred_element_type=jnp.float32) + b2_ref[...]
        vt = jnp.maximum(vt * g2_ref[...] + t2_ref[...], 0.0)
        out_ref[...] = vt


def _vn_call(h, bmat, vn, w1, b1, g1, t1, w2, b2, g2, t2):
    return pl.pallas_call(
        _vn_body,
        grid=(NGRID,),
        in_specs=[
            pl.BlockSpec((NB, D), lambda i: (i, 0)),
            pl.BlockSpec((NB, G), lambda i: (i, 0)),
            pl.BlockSpec((G, D), lambda i: (0, 0)),
            pl.BlockSpec((D, 2 * D), lambda i: (0, 0)),
            pl.BlockSpec((1, 2 * D), lambda i: (0, 0)),
            pl.BlockSpec((1, 2 * D), lambda i: (0, 0)),
            pl.BlockSpec((1, 2 * D), lambda i: (0, 0)),
            pl.BlockSpec((2 * D, D), lambda i: (0, 0)),
            pl.BlockSpec((1, D), lambda i: (0, 0)),
            pl.BlockSpec((1, D), lambda i: (0, 0)),
            pl.BlockSpec((1, D), lambda i: (0, 0)),
        ],
        out_specs=pl.BlockSpec((G, D), lambda i: (0, 0)),
        out_shape=jax.ShapeDtypeStruct((G, D), jnp.float32),
        scratch_shapes=[pltpu.VMEM((G, D), jnp.float32)],
    )(h, bmat, vn, w1, b1, g1, t1, w2, b2, g2, t2)


def _node_body(do_relu, hcur_ref, a0_ref, a1_ref, vnn_ref, bm_ref, eps_ref,
               w1_ref, b1_ref, g1_ref, t1_ref, w2_ref, b2_ref, g2_ref, t2_ref,
               hn_ref, hcn_ref):
    z = eps_ref[0, 0] * hcur_ref[...] + a0_ref[...] + a1_ref[...]
    z = jnp.dot(z, w1_ref[...], preferred_element_type=jnp.float32) + b1_ref[...]
    z = jnp.maximum(z * g1_ref[...] + t1_ref[...], 0.0)
    z = jnp.dot(z, w2_ref[...], preferred_element_type=jnp.float32) + b2_ref[...]
    z = z * g2_ref[...] + t2_ref[...]
    if do_relu:
        z = jnp.maximum(z, 0.0)
    hn_ref[...] = z
    hcn_ref[...] = z + jnp.dot(bm_ref[...], vnn_ref[...],
                               preferred_element_type=jnp.float32)


def _node_call(do_relu, hcur, a0, a1, vnn, bmat, epsp,
               w1, b1, g1, t1, w2, b2, g2, t2):
    import functools
    return pl.pallas_call(
        functools.partial(_node_body, do_relu),
        grid=(NGRID,),
        in_specs=[
            pl.BlockSpec((NB, D), lambda i: (i, 0)),
            pl.BlockSpec((NB, D), lambda i: (i, 0)),
            pl.BlockSpec((NB, D), lambda i: (i, 0)),
            pl.BlockSpec((G, D), lambda i: (0, 0)),
            pl.BlockSpec((NB, G), lambda i: (i, 0)),
            pl.BlockSpec((1, 1), lambda i: (0, 0)),
            pl.BlockSpec((D, 2 * D), lambda i: (0, 0)),
            pl.BlockSpec((1, 2 * D), lambda i: (0, 0)),
            pl.BlockSpec((1, 2 * D), lambda i: (0, 0)),
            pl.BlockSpec((1, 2 * D), lambda i: (0, 0)),
            pl.BlockSpec((2 * D, D), lambda i: (0, 0)),
            pl.BlockSpec((1, D), lambda i: (0, 0)),
            pl.BlockSpec((1, D), lambda i: (0, 0)),
            pl.BlockSpec((1, D), lambda i: (0, 0)),
        ],
        out_specs=[
            pl.BlockSpec((NB, D), lambda i: (i, 0)),
            pl.BlockSpec((NB, D), lambda i: (i, 0)),
        ],
        out_shape=[
            jax.ShapeDtypeStruct((N, D), jnp.float32),
            jax.ShapeDtypeStruct((N, D), jnp.float32),
        ],
    )(hcur, a0, a1, vnn, bmat, epsp, w1, b1, g1, t1, w2, b2, g2, t2)


def _final_body(h_ref, bm_ref, pw_ref, pb_ref, out_ref, accp_ref, accc_ref):
    i = pl.program_id(0)

    @pl.when(i == 0)
    def _():
        accp_ref[...] = jnp.zeros_like(accp_ref)
        accc_ref[...] = jnp.zeros_like(accc_ref)

    bm = bm_ref[...]
    accp_ref[...] += lax.dot_general(bm, h_ref[...], (((0,), (0,)), ((), ())),
                                     preferred_element_type=jnp.float32)
    accc_ref[...] += lax.dot_general(bm, jnp.ones((NB, 8), jnp.float32),
                                     (((0,), (0,)), ((), ())),
                                     preferred_element_type=jnp.float32)

    @pl.when(i == NGRID - 1)
    def _():
        cnt = jnp.maximum(accc_ref[...][:, :1], 1.0)
        hg = accp_ref[...] / cnt
        out_ref[...] = jnp.dot(hg, pw_ref[...],
                               preferred_element_type=jnp.float32) + pb_ref[...]


def _final_call(h, bmat, pw, pb):
    return pl.pallas_call(
        _final_body,
        grid=(NGRID,),
        in_specs=[
            pl.BlockSpec((NB, D), lambda i: (i, 0)),
            pl.BlockSpec((NB, G), lambda i: (i, 0)),
            pl.BlockSpec((D, 128), lambda i: (0, 0)),
            pl.BlockSpec((1, 128), lambda i: (0, 0)),
        ],
        out_specs=pl.BlockSpec((G, 128), lambda i: (0, 0)),
        out_shape=jax.ShapeDtypeStruct((G, 128), jnp.float32),
        scratch_shapes=[pltpu.VMEM((G, D), jnp.float32),
                        pltpu.VMEM((G, 8), jnp.float32)],
    )(h, bmat, pw, pb)


def kernel(x, edge_index, edge_attr, batch, params):
    p = params
    inv = (1.0 + BN_EPS) ** -0.5

    src = edge_index[0]
    dst = edge_index[1]
    ecls = edge_attr[:, 0] + 4 * edge_attr[:, 1] + 16 * edge_attr[:, 2]
    pad = E_PAD - E
    src_p = jnp.concatenate(
        [src, jnp.arange(pad, dtype=jnp.int32) % N]).reshape(
        NW, NBLK, BLK, CH)
    dst_p = jnp.concatenate(
        [dst, N + (jnp.arange(pad, dtype=jnp.int32) % (N_PAD - N))]).reshape(
        NW, NBLK, BLK, CH)
    ecls_p = jnp.concatenate([ecls, jnp.zeros((pad,), jnp.int32)]).reshape(
        NW, NBLK, BLK, CH)
    eidx = jnp.stack([src_p, dst_p, ecls_p], axis=2)

    bflat = p['bond_emb'].reshape(NUM_LAYER * 3 * 8, D)
    vne = p['vn_emb'].reshape(1, D)
    batch2d = batch.reshape(N, 1)

    h, hcur, bmat, etab = _enc_call(x, batch2d, p['atom_emb'], bflat, vne)

    vn = jnp.broadcast_to(p['vn_emb'], (G, D))
    out = None
    for l in range(NUM_LAYER):
        aggp = _edge_agg(hcur, eidx, etab[64 * l:64 * (l + 1)])
        a0 = aggp[0, :N]
        a1 = aggp[1, :N]
        if l < NUM_LAYER - 1:
            vn = _vn_call(
                h, bmat, vn,
                p['vn_W1'][l], p['vn_b1'][l].reshape(1, -1),
                (p['vn_bn1_g'][l] * inv).reshape(1, -1),
                p['vn_bn1_b'][l].reshape(1, -1),
                p['vn_W2'][l], p['vn_b2'][l].reshape(1, -1),
                (p['vn_bn2_g'][l] * inv).reshape(1, -1),
                p['vn_bn2_b'][l].reshape(1, -1),
            )
        else:
            vn = jnp.zeros((G, D), jnp.float32)
        epsp = (1.0 + p['eps'][l]).reshape(1, 1)
        h, hcur = _node_call(
            l < NUM_LAYER - 1, hcur, a0, a1, vn, bmat, epsp,
            p['mlp_W1'][l], p['mlp_b1'][l].reshape(1, -1),
            (p['mlp_bn_g'][l] * inv).reshape(1, -1),
            p['mlp_bn_b'][l].reshape(1, -1),
            p['mlp_W2'][l], p['mlp_b2'][l].reshape(1, -1),
            (p['bn_g'][l] * inv).reshape(1, -1),
            p['bn_b'][l].reshape(1, -1),
        )

    pw = jnp.pad(p['pred_W'], ((0, 0), (0, 128 - NUM_CLASS)))
    pb = jnp.pad(p['pred_b'], (0, 128 - NUM_CLASS)).reshape(1, 128)
    out = _final_call(h, bmat, pw, pb)
    return out[:, :NUM_CLASS]


# speedup vs baseline: 2.5082x; 1.6803x over previous
import jax
import jax.numpy as jnp
from jax import lax
from jax.experimental import pallas as pl
from jax.experimental.pallas import tpu as pltpu
from jax.experimental.pallas import tpu_sc as plsc

N = 10000
E = 320000
D = 128
G = 128
NUM_LAYER = 5
NUM_CLASS = 6
BN_EPS = 1e-5

NB = 400
NGRID = N // NB

NC = 2
NS = 16
LANES = 16
NW = NC * NS
CH = 80
CPT = 126
BLK = 6
NBLK = CPT // BLK
E_PAD = NW * CPT * CH
N_PAD = 10240
RPT = N_PAD // NS
FSTRIP = RPT // CH
RSTRIP = RPT - FSTRIP * CH


def _edge_body(hcur_hbm, eidx_hbm, etab_hbm, out_hbm,
               acc, etab_sp, islab, hbuf0, hbuf1, ebuf0, ebuf1,
               gsem0, gsem1, esem0, esem1, isem):
    c = lax.axis_index("c")
    s = lax.axis_index("s")
    wid = c * NS + s

    @pl.when(s == 0)
    def _():
        pltpu.sync_copy(etab_hbm, etab_sp)

    zeros16 = jnp.zeros((LANES,), jnp.float32)

    def zrow(r, carry):
        for jj in range(D // LANES):
            hbuf0[r, pl.ds(jj * LANES, LANES)] = zeros16
        return carry

    lax.fori_loop(0, CH, zrow, 0)
    base = s * RPT
    for j in range(FSTRIP):
        pltpu.sync_copy(hbuf0, acc.at[pl.ds(base + j * CH, CH)])
    if RSTRIP:
        pltpu.sync_copy(hbuf0.at[pl.ds(0, RSTRIP)],
                        acc.at[pl.ds(base + FSTRIP * CH, RSTRIP)])
    plsc.subcore_barrier()

    pltpu.sync_copy(eidx_hbm.at[wid, 0], islab.at[0])
    pltpu.async_copy(hcur_hbm.at[islab.at[0, 0, 0]], hbuf0, gsem0)
    pltpu.async_copy(etab_sp.at[islab.at[0, 2, 0]], ebuf0, esem0)

    def b_body(b, carry):
        p = b % 2
        q = 1 - p

        def jb2_body(jb2, carry2):
            for k in range(2):
                jb = jb2 * 2 + k
                hbuf = hbuf0 if k == 0 else hbuf1
                ohbuf = hbuf1 if k == 0 else hbuf0
                ebuf = ebuf0 if k == 0 else ebuf1
                oebuf = ebuf1 if k == 0 else ebuf0
                gsem = gsem0 if k == 0 else gsem1
                ogsem = gsem1 if k == 0 else gsem0
                esem = esem0 if k == 0 else esem1
                oesem = esem1 if k == 0 else esem0

                pltpu.make_async_copy(
                    hcur_hbm.at[islab.at[p, 0, jb]], hbuf, gsem).wait()
                pltpu.make_async_copy(
                    etab_sp.at[islab.at[p, 2, jb]], ebuf, esem).wait()

                if k == 0:
                    pltpu.async_copy(
                        hcur_hbm.at[islab.at[p, 0, jb + 1]], ohbuf, ogsem)
                    pltpu.async_copy(
                        etab_sp.at[islab.at[p, 2, jb + 1]], oebuf, oesem)
                else:
                    @pl.when(jb2 < BLK // 2 - 1)
                    def _():
                        pltpu.async_copy(
                            hcur_hbm.at[islab.at[p, 0, jb + 1]], ohbuf, ogsem)
                        pltpu.async_copy(
                            etab_sp.at[islab.at[p, 2, jb + 1]], oebuf, oesem)

                    @pl.when((jb2 == BLK // 2 - 1) & (b + 1 < NBLK))
                    def _():
                        pltpu.make_async_copy(
                            eidx_hbm.at[wid, b + 1], islab.at[q], isem).wait()
                        pltpu.async_copy(
                            hcur_hbm.at[islab.at[q, 0, 0]], ohbuf, ogsem)
                        pltpu.async_copy(
                            etab_sp.at[islab.at[q, 2, 0]], oebuf, oesem)

                    @pl.when((jb2 == 0) & (b + 1 < NBLK))
                    def _():
                        pltpu.async_copy(eidx_hbm.at[wid, b + 1],
                                         islab.at[q], isem)

                @plsc.parallel_loop(0, CH, 1)
                def crow(r):
                    for jj in range(D // LANES):
                        sl = pl.ds(jj * LANES, LANES)
                        hbuf[r, sl] = jnp.maximum(
                            hbuf[r, sl] + ebuf[r, sl], 0.0)

                pltpu.sync_copy(hbuf, acc.at[islab.at[p, 1, jb]], add=True)
            return carry2

        lax.fori_loop(0, BLK // 2, jb2_body, 0)
        return carry

    lax.fori_loop(0, NBLK, b_body, 0)
    plsc.subcore_barrier()
    for j in range(FSTRIP):
        sl = pl.ds(base + j * CH, CH)
        pltpu.sync_copy(acc.at[sl], out_hbm.at[c, sl])
    if RSTRIP:
        sl = pl.ds(base + FSTRIP * CH, RSTRIP)
        pltpu.sync_copy(acc.at[sl], out_hbm.at[c, sl])


_EDGE_CALL_CACHE = []


def _make_edge_call():
    if _EDGE_CALL_CACHE:
        return _EDGE_CALL_CACHE[0]
    call = pl.kernel(
        _edge_body,
        out_type=jax.ShapeDtypeStruct((NC, N_PAD, D), jnp.float32),
        mesh=plsc.VectorSubcoreMesh(core_axis_name="c", subcore_axis_name="s",
                                    num_cores=NC, num_subcores=NS),
        scratch_types=[
            pltpu.VMEM_SHARED((N_PAD, D), jnp.float32),
            pltpu.VMEM_SHARED((64, D), jnp.float32),
            pltpu.VMEM((2, 3, BLK, CH), jnp.int32),
            pltpu.VMEM((CH, D), jnp.float32),
            pltpu.VMEM((CH, D), jnp.float32),
            pltpu.VMEM((CH, D), jnp.float32),
            pltpu.VMEM((CH, D), jnp.float32),
            pltpu.SemaphoreType.DMA,
            pltpu.SemaphoreType.DMA,
            pltpu.SemaphoreType.DMA,
            pltpu.SemaphoreType.DMA,
            pltpu.SemaphoreType.DMA,
        ],
    )
    _EDGE_CALL_CACHE.append(call)
    return call


def _edge_agg(hcur, eidx, etab_l):
    return _make_edge_call()(hcur, eidx, etab_l)


def _enc_body(x_ref, b_ref, aemb_ref, bflat_ref, vne_ref,
              h0_ref, hcur0_ref, bmat_ref, etab_ref):
    i = pl.program_id(0)
    xb = x_ref[...]
    iota64 = lax.broadcasted_iota(jnp.int32, (NB, 64), 1)
    h = jnp.zeros((NB, D), jnp.float32)
    for f in range(9):
        oh = (xb[:, f][:, None] == iota64).astype(jnp.float32)
        h = h + jnp.dot(oh, aemb_ref[f], preferred_element_type=jnp.float32)
    h0_ref[...] = h
    hcur0_ref[...] = h + vne_ref[...]
    bb = b_ref[...]
    iotaG = lax.broadcasted_iota(jnp.int32, (NB, G), 1)
    bmat_ref[...] = (bb == iotaG).astype(jnp.float32)

    @pl.when(i == 0)
    def _():
        r = lax.broadcasted_iota(jnp.int32, (NUM_LAYER * 64, 120), 0)
        col = lax.broadcasted_iota(jnp.int32, (NUM_LAYER * 64, 120), 1)
        lr, cc = r // 64, r % 64
        lc, f, dd = col // 24, (col % 24) // 8, col % 8
        sel = ((lr == lc) & (((cc >> (2 * f)) & 3) == dd)).astype(jnp.float32)
        etab_ref[...] = jnp.dot(sel, bflat_ref[...],
                                preferred_element_type=jnp.float32)


def _enc_call(x, batch2d, aemb, bflat, vne):
    return pl.pallas_call(
        _enc_body,
        grid=(NGRID,),
        in_specs=[
            pl.BlockSpec((NB, 9), lambda i: (i, 0)),
            pl.BlockSpec((NB, 1), lambda i: (i, 0)),
            pl.BlockSpec((9, 64, D), lambda i: (0, 0, 0)),
            pl.BlockSpec((120, D), lambda i: (0, 0)),
            pl.BlockSpec((1, D), lambda i: (0, 0)),
        ],
        out_specs=[
            pl.BlockSpec((NB, D), lambda i: (i, 0)),
            pl.BlockSpec((NB, D), lambda i: (i, 0)),
            pl.BlockSpec((NB, G), lambda i: (i, 0)),
            pl.BlockSpec((NUM_LAYER * 64, D), lambda i: (0, 0)),
        ],
        out_shape=[
            jax.ShapeDtypeStruct((N, D), jnp.float32),
            jax.ShapeDtypeStruct((N, D), jnp.float32),
            jax.ShapeDtypeStruct((N, G), jnp.float32),
            jax.ShapeDtypeStruct((NUM_LAYER * 64, D), jnp.float32),
        ],
    )(x, batch2d, aemb, bflat, vne)


def _vn_body(h_ref, bm_ref, vn_ref, w1_ref, b1_ref, g1_ref, t1_ref,
             w2_ref, b2_ref, g2_ref, t2_ref, out_ref, acc_ref):
    i = pl.program_id(0)

    @pl.when(i == 0)
    def _():
        acc_ref[...] = jnp.zeros_like(acc_ref)

    acc_ref[...] += lax.dot_general(bm_ref[...], h_ref[...],
                                    (((0,), (0,)), ((), ())),
                                    preferred_element_type=jnp.float32)

    @pl.when(i == NGRID - 1)
    def _():
        vt = acc_ref[...] + vn_ref[...]
        vt = jnp.dot(vt, w1_ref[...], preferred_element_type=jnp.float32) + b1_ref[...]
        vt = jnp.maximum(vt * g1_ref[...] + t1_ref[...], 0.0)
        vt = jnp.dot(vt, w2_ref[...], preferred_element_type=jnp.float32) + b2_ref[...]
        vt = jnp.maximum(vt * g2_ref[...] + t2_ref[...], 0.0)
        out_ref[...] = vt


def _vn_call(h, bmat, vn, w1, b1, g1, t1, w2, b2, g2, t2):
    return pl.pallas_call(
        _vn_body,
        grid=(NGRID,),
        in_specs=[
            pl.BlockSpec((NB, D), lambda i: (i, 0)),
            pl.BlockSpec((NB, G), lambda i: (i, 0)),
            pl.BlockSpec((G, D), lambda i: (0, 0)),
            pl.BlockSpec((D, 2 * D), lambda i: (0, 0)),
            pl.BlockSpec((1, 2 * D), lambda i: (0, 0)),
            pl.BlockSpec((1, 2 * D), lambda i: (0, 0)),
            pl.BlockSpec((1, 2 * D), lambda i: (0, 0)),
            pl.BlockSpec((2 * D, D), lambda i: (0, 0)),
            pl.BlockSpec((1, D), lambda i: (0, 0)),
            pl.BlockSpec((1, D), lambda i: (0, 0)),
            pl.BlockSpec((1, D), lambda i: (0, 0)),
        ],
        out_specs=pl.BlockSpec((G, D), lambda i: (0, 0)),
        out_shape=jax.ShapeDtypeStruct((G, D), jnp.float32),
        scratch_shapes=[pltpu.VMEM((G, D), jnp.float32)],
    )(h, bmat, vn, w1, b1, g1, t1, w2, b2, g2, t2)


def _node_body(do_relu, hcur_ref, a0_ref, a1_ref, vnn_ref, bm_ref, eps_ref,
               w1_ref, b1_ref, g1_ref, t1_ref, w2_ref, b2_ref, g2_ref, t2_ref,
               hn_ref, hcn_ref):
    z = eps_ref[0, 0] * hcur_ref[...] + a0_ref[...] + a1_ref[...]
    z = jnp.dot(z, w1_ref[...], preferred_element_type=jnp.float32) + b1_ref[...]
    z = jnp.maximum(z * g1_ref[...] + t1_ref[...], 0.0)
    z = jnp.dot(z, w2_ref[...], preferred_element_type=jnp.float32) + b2_ref[...]
    z = z * g2_ref[...] + t2_ref[...]
    if do_relu:
        z = jnp.maximum(z, 0.0)
    hn_ref[...] = z
    hcn_ref[...] = z + jnp.dot(bm_ref[...], vnn_ref[...],
                               preferred_element_type=jnp.float32)


def _node_call(do_relu, hcur, a0, a1, vnn, bmat, epsp,
               w1, b1, g1, t1, w2, b2, g2, t2):
    import functools
    return pl.pallas_call(
        functools.partial(_node_body, do_relu),
        grid=(NGRID,),
        in_specs=[
            pl.BlockSpec((NB, D), lambda i: (i, 0)),
            pl.BlockSpec((NB, D), lambda i: (i, 0)),
            pl.BlockSpec((NB, D), lambda i: (i, 0)),
            pl.BlockSpec((G, D), lambda i: (0, 0)),
            pl.BlockSpec((NB, G), lambda i: (i, 0)),
            pl.BlockSpec((1, 1), lambda i: (0, 0)),
            pl.BlockSpec((D, 2 * D), lambda i: (0, 0)),
            pl.BlockSpec((1, 2 * D), lambda i: (0, 0)),
            pl.BlockSpec((1, 2 * D), lambda i: (0, 0)),
            pl.BlockSpec((1, 2 * D), lambda i: (0, 0)),
            pl.BlockSpec((2 * D, D), lambda i: (0, 0)),
            pl.BlockSpec((1, D), lambda i: (0, 0)),
            pl.BlockSpec((1, D), lambda i: (0, 0)),
            pl.BlockSpec((1, D), lambda i: (0, 0)),
        ],
        out_specs=[
            pl.BlockSpec((NB, D), lambda i: (i, 0)),
            pl.BlockSpec((NB, D), lambda i: (i, 0)),
        ],
        out_shape=[
            jax.ShapeDtypeStruct((N, D), jnp.float32),
            jax.ShapeDtypeStruct((N, D), jnp.float32),
        ],
    )(hcur, a0, a1, vnn, bmat, epsp, w1, b1, g1, t1, w2, b2, g2, t2)


def _final_body(h_ref, bm_ref, pw_ref, pb_ref, out_ref, accp_ref, accc_ref):
    i = pl.program_id(0)

    @pl.when(i == 0)
    def _():
        accp_ref[...] = jnp.zeros_like(accp_ref)
        accc_ref[...] = jnp.zeros_like(accc_ref)

    bm = bm_ref[...]
    accp_ref[...] += lax.dot_general(bm, h_ref[...], (((0,), (0,)), ((), ())),
                                     preferred_element_type=jnp.float32)
    accc_ref[...] += lax.dot_general(bm, jnp.ones((NB, 8), jnp.float32),
                                     (((0,), (0,)), ((), ())),
                                     preferred_element_type=jnp.float32)

    @pl.when(i == NGRID - 1)
    def _():
        cnt = jnp.maximum(accc_ref[...][:, :1], 1.0)
        hg = accp_ref[...] / cnt
        out_ref[...] = jnp.dot(hg, pw_ref[...],
                               preferred_element_type=jnp.float32) + pb_ref[...]


def _final_call(h, bmat, pw, pb):
    return pl.pallas_call(
        _final_body,
        grid=(NGRID,),
        in_specs=[
            pl.BlockSpec((NB, D), lambda i: (i, 0)),
            pl.BlockSpec((NB, G), lambda i: (i, 0)),
            pl.BlockSpec((D, 128), lambda i: (0, 0)),
            pl.BlockSpec((1, 128), lambda i: (0, 0)),
        ],
        out_specs=pl.BlockSpec((G, 128), lambda i: (0, 0)),
        out_shape=jax.ShapeDtypeStruct((G, 128), jnp.float32),
        scratch_shapes=[pltpu.VMEM((G, D), jnp.float32),
                        pltpu.VMEM((G, 8), jnp.float32)],
    )(h, bmat, pw, pb)


def kernel(x, edge_index, edge_attr, batch, params):
    p = params
    inv = (1.0 + BN_EPS) ** -0.5

    src = edge_index[0]
    dst = edge_index[1]
    ecls = edge_attr[:, 0] + 4 * edge_attr[:, 1] + 16 * edge_attr[:, 2]
    pad = E_PAD - E
    src_p = jnp.concatenate(
        [src, jnp.arange(pad, dtype=jnp.int32) % N]).reshape(
        NW, NBLK, BLK, CH)
    dst_p = jnp.concatenate(
        [dst, N + (jnp.arange(pad, dtype=jnp.int32) % (N_PAD - N))]).reshape(
        NW, NBLK, BLK, CH)
    ecls_p = jnp.concatenate([ecls, jnp.zeros((pad,), jnp.int32)]).reshape(
        NW, NBLK, BLK, CH)
    eidx = jnp.stack([src_p, dst_p, ecls_p], axis=2)

    bflat = p['bond_emb'].reshape(NUM_LAYER * 3 * 8, D)
    vne = p['vn_emb'].reshape(1, D)
    batch2d = batch.reshape(N, 1)

    h, hcur, bmat, etab = _enc_call(x, batch2d, p['atom_emb'], bflat, vne)

    vn = jnp.broadcast_to(p['vn_emb'], (G, D))
    out = None
    for l in range(NUM_LAYER):
        aggp = _edge_agg(hcur, eidx, etab[64 * l:64 * (l + 1)])
        a0 = aggp[0, :N]
        a1 = aggp[1, :N]
        if l < NUM_LAYER - 1:
            vn = _vn_call(
                h, bmat, vn,
                p['vn_W1'][l], p['vn_b1'][l].reshape(1, -1),
                (p['vn_bn1_g'][l] * inv).reshape(1, -1),
                p['vn_bn1_b'][l].reshape(1, -1),
                p['vn_W2'][l], p['vn_b2'][l].reshape(1, -1),
                (p['vn_bn2_g'][l] * inv).reshape(1, -1),
                p['vn_bn2_b'][l].reshape(1, -1),
            )
        else:
            vn = jnp.zeros((G, D), jnp.float32)
        epsp = (1.0 + p['eps'][l]).reshape(1, 1)
        h, hcur = _node_call(
            l < NUM_LAYER - 1, hcur, a0, a1, vn, bmat, epsp,
            p['mlp_W1'][l], p['mlp_b1'][l].reshape(1, -1),
            (p['mlp_bn_g'][l] * inv).reshape(1, -1),
            p['mlp_bn_b'][l].reshape(1, -1),
            p['mlp_W2'][l], p['mlp_b2'][l].reshape(1, -1),
            (p['bn_g'][l] * inv).reshape(1, -1),
            p['bn_b'][l].reshape(1, -1),
        )

    pw = jnp.pad(p['pred_W'], ((0, 0), (0, 128 - NUM_CLASS)))
    pb = jnp.pad(p['pred_b'], (0, 128 - NUM_CLASS)).reshape(1, 128)
    out = _final_call(h, bmat, pw, pb)
    return out[:, :NUM_CLASS]


# speedup vs baseline: 2.5823x; 1.0295x over previous
import jax
import jax.numpy as jnp
from jax import lax
from jax.experimental import pallas as pl
from jax.experimental.pallas import tpu as pltpu
from jax.experimental.pallas import tpu_sc as plsc

N = 10000
E = 320000
D = 128
G = 128
NUM_LAYER = 5
NUM_CLASS = 6
BN_EPS = 1e-5

NB = 400
NGRID = N // NB

NC = 2
NS = 16
LANES = 16
NW = NC * NS
CH = 80
CPT = 126
BLK = 6
NBLK = CPT // BLK
E_PAD = NW * CPT * CH
N_PAD = 10240
RPT = N_PAD // NS
FSTRIP = RPT // CH
RSTRIP = RPT - FSTRIP * CH


def _edge_body(hcur_hbm, eidx_hbm, etab_hbm, out_hbm,
               acc, etab_sp, islab, hbuf0, hbuf1, ebuf0, ebuf1,
               gsem0, gsem1, esem0, esem1, isem):
    c = lax.axis_index("c")
    s = lax.axis_index("s")
    wid = c * NS + s

    @pl.when(s == 0)
    def _():
        pltpu.sync_copy(etab_hbm, etab_sp)

    zeros16 = jnp.zeros((LANES,), jnp.float32)

    def zrow(r, carry):
        for jj in range(D // LANES):
            hbuf0[r, pl.ds(jj * LANES, LANES)] = zeros16
        return carry

    lax.fori_loop(0, CH, zrow, 0)
    base = s * RPT
    for j in range(FSTRIP):
        pltpu.sync_copy(hbuf0, acc.at[pl.ds(base + j * CH, CH)])
    if RSTRIP:
        pltpu.sync_copy(hbuf0.at[pl.ds(0, RSTRIP)],
                        acc.at[pl.ds(base + FSTRIP * CH, RSTRIP)])
    plsc.subcore_barrier()

    pltpu.sync_copy(eidx_hbm.at[wid, 0], islab.at[0])
    pltpu.async_copy(hcur_hbm.at[islab.at[0, 0, 0]], hbuf0, gsem0)
    pltpu.async_copy(etab_sp.at[islab.at[0, 2, 0]], ebuf0, esem0)

    def b_body(b, carry):
        p = b % 2
        q = 1 - p

        def jb2_body(jb2, carry2):
            for k in range(2):
                jb = jb2 * 2 + k
                hbuf = hbuf0 if k == 0 else hbuf1
                ohbuf = hbuf1 if k == 0 else hbuf0
                ebuf = ebuf0 if k == 0 else ebuf1
                oebuf = ebuf1 if k == 0 else ebuf0
                gsem = gsem0 if k == 0 else gsem1
                ogsem = gsem1 if k == 0 else gsem0
                esem = esem0 if k == 0 else esem1
                oesem = esem1 if k == 0 else esem0

                pltpu.make_async_copy(
                    hcur_hbm.at[islab.at[p, 0, jb]], hbuf, gsem).wait()
                pltpu.make_async_copy(
                    etab_sp.at[islab.at[p, 2, jb]], ebuf, esem).wait()

                if k == 0:
                    pltpu.async_copy(
                        hcur_hbm.at[islab.at[p, 0, jb + 1]], ohbuf, ogsem)
                    pltpu.async_copy(
                        etab_sp.at[islab.at[p, 2, jb + 1]], oebuf, oesem)
                else:
                    @pl.when(jb2 < BLK // 2 - 1)
                    def _():
                        pltpu.async_copy(
                            hcur_hbm.at[islab.at[p, 0, jb + 1]], ohbuf, ogsem)
                        pltpu.async_copy(
                            etab_sp.at[islab.at[p, 2, jb + 1]], oebuf, oesem)

                    @pl.when((jb2 == BLK // 2 - 1) & (b + 1 < NBLK))
                    def _():
                        pltpu.make_async_copy(
                            eidx_hbm.at[wid, b + 1], islab.at[q], isem).wait()
                        pltpu.async_copy(
                            hcur_hbm.at[islab.at[q, 0, 0]], ohbuf, ogsem)
                        pltpu.async_copy(
                            etab_sp.at[islab.at[q, 2, 0]], oebuf, oesem)

                    @pl.when((jb2 == 0) & (b + 1 < NBLK))
                    def _():
                        pltpu.async_copy(eidx_hbm.at[wid, b + 1],
                                         islab.at[q], isem)

                @plsc.parallel_loop(0, CH, 1, unroll=4)
                def crow(r):
                    for jj in range(D // LANES):
                        sl = pl.ds(jj * LANES, LANES)
                        hbuf[r, sl] = jnp.maximum(
                            hbuf[r, sl] + ebuf[r, sl], 0.0)

                pltpu.sync_copy(hbuf, acc.at[islab.at[p, 1, jb]], add=True)
            return carry2

        lax.fori_loop(0, BLK // 2, jb2_body, 0)
        return carry

    lax.fori_loop(0, NBLK, b_body, 0)
    plsc.subcore_barrier()
    for j in range(FSTRIP):
        sl = pl.ds(base + j * CH, CH)
        pltpu.sync_copy(acc.at[sl], out_hbm.at[c, sl])
    if RSTRIP:
        sl = pl.ds(base + FSTRIP * CH, RSTRIP)
        pltpu.sync_copy(acc.at[sl], out_hbm.at[c, sl])


_EDGE_CALL_CACHE = []


def _make_edge_call():
    if _EDGE_CALL_CACHE:
        return _EDGE_CALL_CACHE[0]
    call = pl.kernel(
        _edge_body,
        out_type=jax.ShapeDtypeStruct((NC, N_PAD, D), jnp.float32),
        mesh=plsc.VectorSubcoreMesh(core_axis_name="c", subcore_axis_name="s",
                                    num_cores=NC, num_subcores=NS),
        scratch_types=[
            pltpu.VMEM_SHARED((N_PAD, D), jnp.float32),
            pltpu.VMEM_SHARED((64, D), jnp.float32),
            pltpu.VMEM((2, 3, BLK, CH), jnp.int32),
            pltpu.VMEM((CH, D), jnp.float32),
            pltpu.VMEM((CH, D), jnp.float32),
            pltpu.VMEM((CH, D), jnp.float32),
            pltpu.VMEM((CH, D), jnp.float32),
            pltpu.SemaphoreType.DMA,
            pltpu.SemaphoreType.DMA,
            pltpu.SemaphoreType.DMA,
            pltpu.SemaphoreType.DMA,
            pltpu.SemaphoreType.DMA,
        ],
    )
    _EDGE_CALL_CACHE.append(call)
    return call


def _edge_agg(hcur, eidx, etab_l):
    return _make_edge_call()(hcur, eidx, etab_l)


def _enc_body(x_ref, b_ref, aemb_ref, bflat_ref, vne_ref,
              h0_ref, hcur0_ref, bmat_ref, etab_ref):
    i = pl.program_id(0)
    xb = x_ref[...]
    iota64 = lax.broadcasted_iota(jnp.int32, (NB, 64), 1)
    h = jnp.zeros((NB, D), jnp.float32)
    for f in range(9):
        oh = (xb[:, f][:, None] == iota64).astype(jnp.float32)
        h = h + jnp.dot(oh, aemb_ref[f], preferred_element_type=jnp.float32)
    h0_ref[...] = h
    hcur0_ref[...] = h + vne_ref[...]
    bb = b_ref[...]
    iotaG = lax.broadcasted_iota(jnp.int32, (NB, G), 1)
    bmat_ref[...] = (bb == iotaG).astype(jnp.float32)

    @pl.when(i == 0)
    def _():
        r = lax.broadcasted_iota(jnp.int32, (NUM_LAYER * 64, 120), 0)
        col = lax.broadcasted_iota(jnp.int32, (NUM_LAYER * 64, 120), 1)
        lr, cc = r // 64, r % 64
        lc, f, dd = col // 24, (col % 24) // 8, col % 8
        sel = ((lr == lc) & (((cc >> (2 * f)) & 3) == dd)).astype(jnp.float32)
        etab_ref[...] = jnp.dot(sel, bflat_ref[...],
                                preferred_element_type=jnp.float32)


def _enc_call(x, batch2d, aemb, bflat, vne):
    return pl.pallas_call(
        _enc_body,
        grid=(NGRID,),
        in_specs=[
            pl.BlockSpec((NB, 9), lambda i: (i, 0)),
            pl.BlockSpec((NB, 1), lambda i: (i, 0)),
            pl.BlockSpec((9, 64, D), lambda i: (0, 0, 0)),
            pl.BlockSpec((120, D), lambda i: (0, 0)),
            pl.BlockSpec((1, D), lambda i: (0, 0)),
        ],
        out_specs=[
            pl.BlockSpec((NB, D), lambda i: (i, 0)),
            pl.BlockSpec((NB, D), lambda i: (i, 0)),
            pl.BlockSpec((NB, G), lambda i: (i, 0)),
            pl.BlockSpec((NUM_LAYER * 64, D), lambda i: (0, 0)),
        ],
        out_shape=[
            jax.ShapeDtypeStruct((N, D), jnp.float32),
            jax.ShapeDtypeStruct((N, D), jnp.float32),
            jax.ShapeDtypeStruct((N, G), jnp.float32),
            jax.ShapeDtypeStruct((NUM_LAYER * 64, D), jnp.float32),
        ],
    )(x, batch2d, aemb, bflat, vne)


def _vn_body(h_ref, bm_ref, vn_ref, w1_ref, b1_ref, g1_ref, t1_ref,
             w2_ref, b2_ref, g2_ref, t2_ref, out_ref, acc_ref):
    i = pl.program_id(0)

    @pl.when(i == 0)
    def _():
        acc_ref[...] = jnp.zeros_like(acc_ref)

    acc_ref[...] += lax.dot_general(bm_ref[...], h_ref[...],
                                    (((0,), (0,)), ((), ())),
                                    preferred_element_type=jnp.float32)

    @pl.when(i == NGRID - 1)
    def _():
        vt = acc_ref[...] + vn_ref[...]
        vt = jnp.dot(vt, w1_ref[...], preferred_element_type=jnp.float32) + b1_ref[...]
        vt = jnp.maximum(vt * g1_ref[...] + t1_ref[...], 0.0)
        vt = jnp.dot(vt, w2_ref[...], preferred_element_type=jnp.float32) + b2_ref[...]
        vt = jnp.maximum(vt * g2_ref[...] + t2_ref[...], 0.0)
        out_ref[...] = vt


def _vn_call(h, bmat, vn, w1, b1, g1, t1, w2, b2, g2, t2):
    return pl.pallas_call(
        _vn_body,
        grid=(NGRID,),
        in_specs=[
            pl.BlockSpec((NB, D), lambda i: (i, 0)),
            pl.BlockSpec((NB, G), lambda i: (i, 0)),
            pl.BlockSpec((G, D), lambda i: (0, 0)),
            pl.BlockSpec((D, 2 * D), lambda i: (0, 0)),
            pl.BlockSpec((1, 2 * D), lambda i: (0, 0)),
            pl.BlockSpec((1, 2 * D), lambda i: (0, 0)),
            pl.BlockSpec((1, 2 * D), lambda i: (0, 0)),
            pl.BlockSpec((2 * D, D), lambda i: (0, 0)),
            pl.BlockSpec((1, D), lambda i: (0, 0)),
            pl.BlockSpec((1, D), lambda i: (0, 0)),
            pl.BlockSpec((1, D), lambda i: (0, 0)),
        ],
        out_specs=pl.BlockSpec((G, D), lambda i: (0, 0)),
        out_shape=jax.ShapeDtypeStruct((G, D), jnp.float32),
        scratch_shapes=[pltpu.VMEM((G, D), jnp.float32)],
    )(h, bmat, vn, w1, b1, g1, t1, w2, b2, g2, t2)


def _node_body(do_relu, hcur_ref, a0_ref, a1_ref, vnn_ref, bm_ref, eps_ref,
               w1_ref, b1_ref, g1_ref, t1_ref, w2_ref, b2_ref, g2_ref, t2_ref,
               hn_ref, hcn_ref):
    z = eps_ref[0, 0] * hcur_ref[...] + a0_ref[...] + a1_ref[...]
    z = jnp.dot(z, w1_ref[...], preferred_element_type=jnp.float32) + b1_ref[...]
    z = jnp.maximum(z * g1_ref[...] + t1_ref[...], 0.0)
    z = jnp.dot(z, w2_ref[...], preferred_element_type=jnp.float32) + b2_ref[...]
    z = z * g2_ref[...] + t2_ref[...]
    if do_relu:
        z = jnp.maximum(z, 0.0)
    hn_ref[...] = z
    hcn_ref[...] = z + jnp.dot(bm_ref[...], vnn_ref[...],
                               preferred_element_type=jnp.float32)


def _node_call(do_relu, hcur, a0, a1, vnn, bmat, epsp,
               w1, b1, g1, t1, w2, b2, g2, t2):
    import functools
    return pl.pallas_call(
        functools.partial(_node_body, do_relu),
        grid=(NGRID,),
        in_specs=[
            pl.BlockSpec((NB, D), lambda i: (i, 0)),
            pl.BlockSpec((NB, D), lambda i: (i, 0)),
            pl.BlockSpec((NB, D), lambda i: (i, 0)),
            pl.BlockSpec((G, D), lambda i: (0, 0)),
            pl.BlockSpec((NB, G), lambda i: (i, 0)),
            pl.BlockSpec((1, 1), lambda i: (0, 0)),
            pl.BlockSpec((D, 2 * D), lambda i: (0, 0)),
            pl.BlockSpec((1, 2 * D), lambda i: (0, 0)),
            pl.BlockSpec((1, 2 * D), lambda i: (0, 0)),
            pl.BlockSpec((1, 2 * D), lambda i: (0, 0)),
            pl.BlockSpec((2 * D, D), lambda i: (0, 0)),
            pl.BlockSpec((1, D), lambda i: (0, 0)),
            pl.BlockSpec((1, D), lambda i: (0, 0)),
            pl.BlockSpec((1, D), lambda i: (0, 0)),
        ],
        out_specs=[
            pl.BlockSpec((NB, D), lambda i: (i, 0)),
            pl.BlockSpec((NB, D), lambda i: (i, 0)),
        ],
        out_shape=[
            jax.ShapeDtypeStruct((N, D), jnp.float32),
            jax.ShapeDtypeStruct((N, D), jnp.float32),
        ],
    )(hcur, a0, a1, vnn, bmat, epsp, w1, b1, g1, t1, w2, b2, g2, t2)


def _final_body(h_ref, bm_ref, pw_ref, pb_ref, out_ref, accp_ref, accc_ref):
    i = pl.program_id(0)

    @pl.when(i == 0)
    def _():
        accp_ref[...] = jnp.zeros_like(accp_ref)
        accc_ref[...] = jnp.zeros_like(accc_ref)

    bm = bm_ref[...]
    accp_ref[...] += lax.dot_general(bm, h_ref[...], (((0,), (0,)), ((), ())),
                                     preferred_element_type=jnp.float32)
    accc_ref[...] += lax.dot_general(bm, jnp.ones((NB, 8), jnp.float32),
                                     (((0,), (0,)), ((), ())),
                                     preferred_element_type=jnp.float32)

    @pl.when(i == NGRID - 1)
    def _():
        cnt = jnp.maximum(accc_ref[...][:, :1], 1.0)
        hg = accp_ref[...] / cnt
        out_ref[...] = jnp.dot(hg, pw_ref[...],
                               preferred_element_type=jnp.float32) + pb_ref[...]


def _final_call(h, bmat, pw, pb):
    return pl.pallas_call(
        _final_body,
        grid=(NGRID,),
        in_specs=[
            pl.BlockSpec((NB, D), lambda i: (i, 0)),
            pl.BlockSpec((NB, G), lambda i: (i, 0)),
            pl.BlockSpec((D, 128), lambda i: (0, 0)),
            pl.BlockSpec((1, 128), lambda i: (0, 0)),
        ],
        out_specs=pl.BlockSpec((G, 128), lambda i: (0, 0)),
        out_shape=jax.ShapeDtypeStruct((G, 128), jnp.float32),
        scratch_shapes=[pltpu.VMEM((G, D), jnp.float32),
                        pltpu.VMEM((G, 8), jnp.float32)],
    )(h, bmat, pw, pb)


def kernel(x, edge_index, edge_attr, batch, params):
    p = params
    inv = (1.0 + BN_EPS) ** -0.5

    src = edge_index[0]
    dst = edge_index[1]
    ecls = edge_attr[:, 0] + 4 * edge_attr[:, 1] + 16 * edge_attr[:, 2]
    pad = E_PAD - E
    src_p = jnp.concatenate(
        [src, jnp.arange(pad, dtype=jnp.int32) % N]).reshape(
        NW, NBLK, BLK, CH)
    dst_p = jnp.concatenate(
        [dst, N + (jnp.arange(pad, dtype=jnp.int32) % (N_PAD - N))]).reshape(
        NW, NBLK, BLK, CH)
    ecls_p = jnp.concatenate([ecls, jnp.zeros((pad,), jnp.int32)]).reshape(
        NW, NBLK, BLK, CH)
    eidx = jnp.stack([src_p, dst_p, ecls_p], axis=2)

    bflat = p['bond_emb'].reshape(NUM_LAYER * 3 * 8, D)
    vne = p['vn_emb'].reshape(1, D)
    batch2d = batch.reshape(N, 1)

    h, hcur, bmat, etab = _enc_call(x, batch2d, p['atom_emb'], bflat, vne)

    vn = jnp.broadcast_to(p['vn_emb'], (G, D))
    out = None
    for l in range(NUM_LAYER):
        aggp = _edge_agg(hcur, eidx, etab[64 * l:64 * (l + 1)])
        a0 = aggp[0, :N]
        a1 = aggp[1, :N]
        if l < NUM_LAYER - 1:
            vn = _vn_call(
                h, bmat, vn,
                p['vn_W1'][l], p['vn_b1'][l].reshape(1, -1),
                (p['vn_bn1_g'][l] * inv).reshape(1, -1),
                p['vn_bn1_b'][l].reshape(1, -1),
                p['vn_W2'][l], p['vn_b2'][l].reshape(1, -1),
                (p['vn_bn2_g'][l] * inv).reshape(1, -1),
                p['vn_bn2_b'][l].reshape(1, -1),
            )
        else:
            vn = jnp.zeros((G, D), jnp.float32)
        epsp = (1.0 + p['eps'][l]).reshape(1, 1)
        h, hcur = _node_call(
            l < NUM_LAYER - 1, hcur, a0, a1, vn, bmat, epsp,
            p['mlp_W1'][l], p['mlp_b1'][l].reshape(1, -1),
            (p['mlp_bn_g'][l] * inv).reshape(1, -1),
            p['mlp_bn_b'][l].reshape(1, -1),
            p['mlp_W2'][l], p['mlp_b2'][l].reshape(1, -1),
            (p['bn_g'][l] * inv).reshape(1, -1),
            p['bn_b'][l].reshape(1, -1),
        )

    pw = jnp.pad(p['pred_W'], ((0, 0), (0, 128 - NUM_CLASS)))
    pb = jnp.pad(p['pred_b'], (0, 128 - NUM_CLASS)).reshape(1, 128)
    out = _final_call(h, bmat, pw, pb)
    return out[:, :NUM_CLASS]
